# baseline (speedup 1.0000x reference)
import functools

import jax
import jax.numpy as jnp
from jax.experimental import pallas as pl
from jax.experimental.pallas import tpu as pltpu

T = 2048
D = 768
H = 4 * D
E = 8
TOP_K = 2

TILE_T = 256


def _moe_body(x_ref, xb_ref, wr_ref, w1_ref, b1_ref, w2_ref, b2_ref,
              gamma_ref, beta_ref, gs_ref, out_ref, comb_ref, acc_ref):
    e = pl.program_id(1)

    @pl.when(e == 0)
    def _router():
        x = x_ref[...]
        logits = jnp.dot(x, wr_ref[...], preferred_element_type=jnp.float32)
        m = jnp.max(logits, axis=-1, keepdims=True)
        el = jnp.exp(logits - m)
        probs = el / jnp.sum(el, axis=-1, keepdims=True)
        idx = jax.lax.broadcasted_iota(jnp.int32, probs.shape, 1)
        p1 = jnp.max(probs, axis=-1, keepdims=True)
        i1 = jnp.min(jnp.where(probs == p1, idx, E), axis=-1, keepdims=True)
        oh1 = (idx == i1)
        pm = jnp.where(oh1, -1.0, probs)
        p2 = jnp.max(pm, axis=-1, keepdims=True)
        i2 = jnp.min(jnp.where(pm == p2, idx, E), axis=-1, keepdims=True)
        oh2 = (idx == i2)
        comb_ref[...] = p1 * oh1.astype(jnp.float32) + p2 * oh2.astype(jnp.float32)
        acc_ref[...] = jnp.zeros_like(acc_ref)

    xb = xb_ref[...]
    h = jnp.dot(xb, w1_ref[0], preferred_element_type=jnp.float32)
    h = h + b1_ref[0]
    h = h * (1.0 / (1.0 + jnp.exp(-h)))
    eo = jnp.dot(h.astype(jnp.bfloat16), w2_ref[0],
                 preferred_element_type=jnp.float32)
    eo = eo + b2_ref[0]
    comb = comb_ref[...]
    lane = jax.lax.broadcasted_iota(jnp.int32, comb.shape, 1)
    w_col = jnp.sum(jnp.where(lane == e, comb, 0.0), axis=-1, keepdims=True)
    acc_ref[...] += w_col * eo

    @pl.when(e == E - 1)
    def _finish():
        y = x_ref[...] + acc_ref[...] * gs_ref[0]
        mu = jnp.mean(y, axis=-1, keepdims=True)
        yc = y - mu
        var = jnp.mean(yc * yc, axis=-1, keepdims=True)
        out_ref[...] = yc * jax.lax.rsqrt(var + 1e-5) * gamma_ref[...] + beta_ref[...]


@jax.jit
def kernel(x, W_r, W1, b1, W2, b2, gamma, beta, gate_scale):
    xb = x.astype(jnp.bfloat16)
    W1b = W1.astype(jnp.bfloat16)
    W2b = W2.astype(jnp.bfloat16)
    n_t = T // TILE_T

    grid = (n_t, E)
    out = pl.pallas_call(
        _moe_body,
        grid=grid,
        in_specs=[
            pl.BlockSpec((TILE_T, D), lambda t, e: (t, 0)),
            pl.BlockSpec((TILE_T, D), lambda t, e: (t, 0)),
            pl.BlockSpec((D, E), lambda t, e: (0, 0)),
            pl.BlockSpec((1, D, H), lambda t, e: (e, 0, 0)),
            pl.BlockSpec((1, 1, H), lambda t, e: (e, 0, 0)),
            pl.BlockSpec((1, H, D), lambda t, e: (e, 0, 0)),
            pl.BlockSpec((1, 1, D), lambda t, e: (e, 0, 0)),
            pl.BlockSpec((1, D), lambda t, e: (0, 0)),
            pl.BlockSpec((1, D), lambda t, e: (0, 0)),
            pl.BlockSpec(memory_space=pltpu.SMEM),
        ],
        out_specs=pl.BlockSpec((TILE_T, D), lambda t, e: (t, 0)),
        out_shape=jax.ShapeDtypeStruct((T, D), jnp.float32),
        scratch_shapes=[
            pltpu.VMEM((TILE_T, E), jnp.float32),
            pltpu.VMEM((TILE_T, D), jnp.float32),
        ],
    )(x, xb, W_r, W1b, b1.reshape(E, 1, H), W2b, b2.reshape(E, 1, D),
      gamma.reshape(1, D), beta.reshape(1, D),
      gate_scale.reshape(1))
    return out
